# baseline (speedup 1.0000x reference)
import functools

import jax
import jax.numpy as jnp
from jax import lax
from jax.experimental import pallas as pl
from jax.experimental.pallas import tpu as pltpu
from jax.experimental.pallas import tpu_sc as plsc

N = 100000
E = 6400000
TRIALS = 10

NPAD = 100352
ACOLS = NPAD // 8
TPAD = 100008
NACC = 100096
CHUNK = 2048
NC, NS = 2, 16
NW = NC * NS
NCHUNK = E // CHUNK
KPW = -(-NCHUNK // NW)
SLICE = NACC // NS


def _payoff_body(b_ref, p_ref, u_ref, o_ref):
    mask = b_ref[...] > 0.5
    cnt = jnp.sum((u_ref[...] < p_ref[...]).astype(jnp.float32),
                  axis=0, keepdims=True)
    o_ref[...] = jnp.where(mask, cnt + 16.0, 0.0)


def _packed_table(belief2, probs2, u_t):
    return pl.pallas_call(
        _payoff_body,
        grid=(NPAD // ACOLS,),
        in_specs=[
            pl.BlockSpec((1, ACOLS), lambda i: (0, i)),
            pl.BlockSpec((1, ACOLS), lambda i: (0, i)),
            pl.BlockSpec((TRIALS, ACOLS), lambda i: (0, i)),
        ],
        out_specs=pl.BlockSpec((1, ACOLS), lambda i: (0, i)),
        out_shape=jax.ShapeDtypeStruct((1, NPAD), jnp.float32),
    )(belief2, probs2, u_t)


_PIECES = ((0, 2048), (2048, 2048), (4096, 2048), (6144, SLICE - 6144))


def _sc_body(tbl_hbm, src_hbm, dst_hbm, p0_hbm, p1_hbm,
             tbl_v, srcb, dstb, msg0, msg1, stage_v,
             sem_in, sem_sc, acc0_sh, acc1_sh):
    cid = lax.axis_index("c")
    sid = lax.axis_index("s")
    w = sid * NC + cid
    base = w * KPW

    pltpu.sync_copy(tbl_hbm.at[pl.ds(0, TPAD)], tbl_v)

    def _zero(i, _):
        stage_v[pl.ds(i * 16, 16)] = jnp.zeros((16,), jnp.float32)
        return 0
    lax.fori_loop(0, 2048 // 16, _zero, 0)
    for off, sz in _PIECES:
        psl = pl.ds(sid * SLICE + off, sz)
        pltpu.sync_copy(stage_v.at[pl.ds(0, sz)], acc0_sh.at[psl])
        pltpu.sync_copy(stage_v.at[pl.ds(0, sz)], acc1_sh.at[psl])
    plsc.subcore_barrier()

    def _issue_in(g, b):
        pltpu.async_copy(src_hbm.at[g], srcb[b], sem_in[b])
        pltpu.async_copy(dst_hbm.at[g], dstb[b], sem_in[b])

    def _wait_in(g, b):
        pltpu.make_async_copy(src_hbm.at[g], srcb[b], sem_in[b]).wait()
        pltpu.make_async_copy(dst_hbm.at[g], dstb[b], sem_in[b]).wait()

    def _issue_sc(b):
        pltpu.async_copy(msg0[b], acc0_sh.at[dstb[b]], sem_sc[b], add=True)
        pltpu.async_copy(msg1[b], acc1_sh.at[dstb[b]], sem_sc[b], add=True)

    def _wait_sc(b):
        pltpu.make_async_copy(msg0[b], acc0_sh.at[dstb[b]], sem_sc[b]).wait()
        pltpu.make_async_copy(msg1[b], acc1_sh.at[dstb[b]], sem_sc[b]).wait()

    def _compute(b):
        for i in range(16):
            for j in range(8):
                s = srcb[b][i, pl.ds(j * 16, 16)]
                v = plsc.load_gather(tbl_v, [s])
                m = v >= 16.0
                o = i * 128 + j * 16
                msg0[b][pl.ds(o, 16)] = v
                msg1[b][pl.ds(o, 16)] = jnp.where(m, 1.0, 0.0)

    _issue_in(base, 0)

    def _pair(kk2, _):
        for b in (0, 1):
            k = 2 * kk2 + b
            g = base + k

            @pl.when(g < NCHUNK)
            def _():
                _wait_in(g, b)
                _compute(b)

            @pl.when((k >= 1) & (g - 1 < NCHUNK))
            def _():
                _wait_sc(1 - b)

            @pl.when((k <= KPW - 2) & (g + 1 < NCHUNK))
            def _():
                _issue_in(g + 1, 1 - b)

            @pl.when(g < NCHUNK)
            def _():
                _issue_sc(b)
        return 0

    lax.fori_loop(0, KPW // 2, _pair, 0)

    @pl.when(base + KPW - 1 < NCHUNK)
    def _():
        _wait_sc(1)

    plsc.subcore_barrier()

    for off, sz in _PIECES:
        psl = pl.ds(sid * SLICE + off, sz)
        osl = pl.ds(cid * NACC + sid * SLICE + off, sz)
        st = stage_v.at[pl.ds(0, sz)]
        pltpu.sync_copy(acc0_sh.at[psl], st)
        pltpu.sync_copy(st, p0_hbm.at[osl])
        pltpu.sync_copy(acc1_sh.at[psl], st)
        pltpu.sync_copy(st, p1_hbm.at[osl])


def _sc_scatter(tbl, src3, dst3):
    mesh = plsc.VectorSubcoreMesh(core_axis_name="c", subcore_axis_name="s")
    return pl.kernel(
        _sc_body,
        out_type=(
            jax.ShapeDtypeStruct((NC * NACC,), jnp.float32),
            jax.ShapeDtypeStruct((NC * NACC,), jnp.float32),
        ),
        mesh=mesh,
        scratch_types=(
            pltpu.VMEM((TPAD,), jnp.float32),
            [pltpu.VMEM((16, 128), jnp.int32) for _ in range(2)],
            [pltpu.VMEM((CHUNK,), jnp.int32) for _ in range(2)],
            [pltpu.VMEM((CHUNK,), jnp.float32) for _ in range(2)],
            [pltpu.VMEM((CHUNK,), jnp.float32) for _ in range(2)],
            pltpu.VMEM((2048,), jnp.float32),
            [pltpu.SemaphoreType.DMA for _ in range(2)],
            [pltpu.SemaphoreType.DMA for _ in range(2)],
            pltpu.VMEM_SHARED((NACC,), jnp.float32),
            pltpu.VMEM_SHARED((NACC,), jnp.float32),
        ),
        compiler_params=pltpu.CompilerParams(needs_layout_passes=False),
    )(tbl, src3, dst3)


def _merge_body(p0_ref, p1_ref, o_ref):
    vsum = p0_ref[0, :] + p0_ref[1, :]
    cnt = p1_ref[0, :] + p1_ref[1, :]
    o_ref[0, :] = vsum - 16.0 * cnt
    o_ref[1, :] = 10.0 * cnt


def _merge(p0, p1):
    return pl.pallas_call(
        _merge_body,
        out_shape=jax.ShapeDtypeStruct((2, NACC), jnp.float32),
    )(p0, p1)


@jax.jit
def kernel(belief, probs, bernoulli_uniforms, edge_index):
    pad = NPAD - N
    belief2 = jnp.pad(belief, (0, pad)).reshape(1, NPAD)
    probs2 = jnp.pad(probs, (0, pad)).reshape(1, NPAD)
    u_t = jnp.pad(bernoulli_uniforms.T, ((0, 0), (0, pad)))

    tbl = _packed_table(belief2, probs2, u_t).reshape(NPAD)

    src3 = edge_index[0].reshape(NCHUNK, 16, 128)
    dst3 = edge_index[1].reshape(NCHUNK, CHUNK)
    p0, p1 = _sc_scatter(tbl, src3, dst3)

    merged = _merge(p0.reshape(NC, NACC), p1.reshape(NC, NACC))
    return merged[:, :N].T


# speedup vs baseline: 1.5991x; 1.5991x over previous
import functools

import jax
import jax.numpy as jnp
from jax import lax
from jax.experimental import pallas as pl
from jax.experimental.pallas import tpu as pltpu
from jax.experimental.pallas import tpu_sc as plsc

N = 100000
E = 6400000
TRIALS = 10

NPAD = 100352
ACOLS = NPAD // 8
TPAD = 100008
NACC = 100096
CHUNK = 2048
NC, NS = 2, 16
NW = NC * NS
REAL = E // CHUNK
KPW = 98
NCHUNK = NW * KPW
EPAD = NCHUNK * CHUNK
SLICE = NACC // NS


def _payoff_body(b_ref, p_ref, u_ref, o_ref):
    mask = b_ref[...] > 0.5
    cnt = jnp.sum((u_ref[...] < p_ref[...]).astype(jnp.float32),
                  axis=0, keepdims=True)
    o_ref[...] = jnp.where(mask, cnt + 16.0, 0.0)


def _packed_table(belief2, probs2, u_t):
    return pl.pallas_call(
        _payoff_body,
        grid=(NPAD // ACOLS,),
        in_specs=[
            pl.BlockSpec((1, ACOLS), lambda i: (0, i)),
            pl.BlockSpec((1, ACOLS), lambda i: (0, i)),
            pl.BlockSpec((TRIALS, ACOLS), lambda i: (0, i)),
        ],
        out_specs=pl.BlockSpec((1, ACOLS), lambda i: (0, i)),
        out_shape=jax.ShapeDtypeStruct((1, NPAD), jnp.float32),
    )(belief2, probs2, u_t)


_PIECES = ((0, 2048), (2048, 2048), (4096, 2048), (6144, SLICE - 6144))


def _sc_body(tbl_hbm, src_hbm, dst_hbm, p0_hbm, p1_hbm,
             tbl_v, srcb, dstb, msg0, msg1, stage_v,
             sem_in, sem_sc, acc0_sh, acc1_sh):
    cid = lax.axis_index("c")
    sid = lax.axis_index("s")
    w = sid * NC + cid
    base = w * KPW

    pltpu.sync_copy(tbl_hbm.at[pl.ds(0, TPAD)], tbl_v)

    def _zero(i, _):
        stage_v[pl.ds(i * 16, 16)] = jnp.zeros((16,), jnp.float32)
        return 0
    lax.fori_loop(0, 2048 // 16, _zero, 0)
    for off, sz in _PIECES:
        psl = pl.ds(sid * SLICE + off, sz)
        pltpu.sync_copy(stage_v.at[pl.ds(0, sz)], acc0_sh.at[psl])
        pltpu.sync_copy(stage_v.at[pl.ds(0, sz)], acc1_sh.at[psl])
    plsc.subcore_barrier()

    def _issue_in(g, b):
        pltpu.async_copy(src_hbm.at[g], srcb[b], sem_in[b])
        pltpu.async_copy(dst_hbm.at[g], dstb[b], sem_in[b])

    def _wait_in(g, b):
        pltpu.make_async_copy(src_hbm.at[g], srcb[b], sem_in[b]).wait()
        pltpu.make_async_copy(dst_hbm.at[g], dstb[b], sem_in[b]).wait()

    def _issue_sc(b):
        pltpu.async_copy(msg0[b], acc0_sh.at[dstb[b]], sem_sc[b], add=True)
        pltpu.async_copy(msg1[b], acc1_sh.at[dstb[b]], sem_sc[b], add=True)

    def _wait_sc(b):
        pltpu.make_async_copy(msg0[b], acc0_sh.at[dstb[b]], sem_sc[b]).wait()
        pltpu.make_async_copy(msg1[b], acc1_sh.at[dstb[b]], sem_sc[b]).wait()

    def _compute(b):
        for i in range(16):
            for j in range(8):
                s = srcb[b][i, pl.ds(j * 16, 16)]
                v = plsc.load_gather(tbl_v, [s])
                m = v >= 16.0
                o = i * 128 + j * 16
                msg0[b][pl.ds(o, 16)] = v
                msg1[b][pl.ds(o, 16)] = jnp.where(m, 1.0, 0.0)

    _issue_in(base, 0)

    def _pair(kk2, _):
        for b in (0, 1):
            k = 2 * kk2 + b
            g = base + k
            _wait_in(g, b)
            _compute(b)

            @pl.when(k >= 1)
            def _():
                _wait_sc(1 - b)

            @pl.when(k <= KPW - 2)
            def _():
                _issue_in(g + 1, 1 - b)

            _issue_sc(b)
        return 0

    lax.fori_loop(0, KPW // 2, _pair, 0)
    _wait_sc(1)
    plsc.subcore_barrier()

    for off, sz in _PIECES:
        psl = pl.ds(sid * SLICE + off, sz)
        osl = pl.ds(cid * NACC + sid * SLICE + off, sz)
        st = stage_v.at[pl.ds(0, sz)]
        pltpu.sync_copy(acc0_sh.at[psl], st)
        pltpu.sync_copy(st, p0_hbm.at[osl])
        pltpu.sync_copy(acc1_sh.at[psl], st)
        pltpu.sync_copy(st, p1_hbm.at[osl])


def _sc_scatter(tbl, src3, dst3):
    mesh = plsc.VectorSubcoreMesh(core_axis_name="c", subcore_axis_name="s")
    return pl.kernel(
        _sc_body,
        out_type=(
            jax.ShapeDtypeStruct((NC * NACC,), jnp.float32),
            jax.ShapeDtypeStruct((NC * NACC,), jnp.float32),
        ),
        mesh=mesh,
        scratch_types=(
            pltpu.VMEM((TPAD,), jnp.float32),
            [pltpu.VMEM((16, 128), jnp.int32) for _ in range(2)],
            [pltpu.VMEM((CHUNK,), jnp.int32) for _ in range(2)],
            [pltpu.VMEM((CHUNK,), jnp.float32) for _ in range(2)],
            [pltpu.VMEM((CHUNK,), jnp.float32) for _ in range(2)],
            pltpu.VMEM((2048,), jnp.float32),
            [pltpu.SemaphoreType.DMA for _ in range(2)],
            [pltpu.SemaphoreType.DMA for _ in range(2)],
            pltpu.VMEM_SHARED((NACC,), jnp.float32),
            pltpu.VMEM_SHARED((NACC,), jnp.float32),
        ),
        compiler_params=pltpu.CompilerParams(needs_layout_passes=False),
    )(tbl, src3, dst3)


def _merge_body(p0_ref, p1_ref, o_ref):
    vsum = p0_ref[0, :] + p0_ref[1, :]
    cnt = p1_ref[0, :] + p1_ref[1, :]
    o_ref[0, :] = vsum - 16.0 * cnt
    o_ref[1, :] = 10.0 * cnt


def _merge(p0, p1):
    return pl.pallas_call(
        _merge_body,
        out_shape=jax.ShapeDtypeStruct((2, NACC), jnp.float32),
    )(p0, p1)


@jax.jit
def kernel(belief, probs, bernoulli_uniforms, edge_index):
    pad = NPAD - N
    belief2 = jnp.pad(belief, (0, pad)).reshape(1, NPAD)
    probs2 = jnp.pad(probs, (0, pad)).reshape(1, NPAD)
    u_t = jnp.pad(bernoulli_uniforms.T, ((0, 0), (0, pad)))

    tbl = _packed_table(belief2, probs2, u_t).reshape(NPAD)

    epad = EPAD - E
    src_pad = jnp.full((epad,), N, dtype=jnp.int32)
    dst_pad = (jnp.arange(epad, dtype=jnp.int32) * 521) % N
    src3 = jnp.concatenate([edge_index[0], src_pad]).reshape(NCHUNK, 16, 128)
    dst3 = jnp.concatenate([edge_index[1], dst_pad]).reshape(NCHUNK, CHUNK)
    p0, p1 = _sc_scatter(tbl, src3, dst3)

    merged = _merge(p0.reshape(NC, NACC), p1.reshape(NC, NACC))
    return merged[:, :N].T


# speedup vs baseline: 2.1919x; 1.3707x over previous
import functools

import jax
import jax.numpy as jnp
from jax import lax
from jax.experimental import pallas as pl
from jax.experimental.pallas import tpu as pltpu
from jax.experimental.pallas import tpu_sc as plsc

N = 100000
E = 6400000
TRIALS = 10

NPAD = 100352
ACOLS = NPAD // 8
TPAD = 100008
NACC = 100096
CHUNK = 2048
NC, NS = 2, 16
NW = NC * NS
REAL = E // CHUNK
KPW = 98
NCHUNK = NW * KPW
EPAD = NCHUNK * CHUNK
SLICE = NACC // NS


def _payoff_body(b_ref, p_ref, u_ref, o_ref):
    mask = b_ref[...] > 0.5
    cnt = jnp.sum((u_ref[...] < p_ref[...]).astype(jnp.float32),
                  axis=0, keepdims=True)
    o_ref[...] = jnp.where(mask, cnt + 16.0, 0.0)


def _packed_table(belief2, probs2, u_t):
    return pl.pallas_call(
        _payoff_body,
        grid=(NPAD // ACOLS,),
        in_specs=[
            pl.BlockSpec((1, ACOLS), lambda i: (0, i)),
            pl.BlockSpec((1, ACOLS), lambda i: (0, i)),
            pl.BlockSpec((TRIALS, ACOLS), lambda i: (0, i)),
        ],
        out_specs=pl.BlockSpec((1, ACOLS), lambda i: (0, i)),
        out_shape=jax.ShapeDtypeStruct((1, NPAD), jnp.float32),
    )(belief2, probs2, u_t)


_PIECES = ((0, 2048), (2048, 2048), (4096, 2048), (6144, SLICE - 6144))


def _sc_body(tbl_hbm, edge_hbm, p0_hbm, p1_hbm,
             tbl_v, srcb, dstb, msg0, msg1, stage_v,
             sem_in, sem_sc, acc0_sh, acc1_sh):
    cid = lax.axis_index("c")
    sid = lax.axis_index("s")
    w = sid * NC + cid
    base = w * KPW

    pltpu.sync_copy(tbl_hbm.at[pl.ds(0, TPAD)], tbl_v)

    def _zero(i, _):
        stage_v[pl.ds(i * 16, 16)] = jnp.zeros((16,), jnp.float32)
        return 0
    lax.fori_loop(0, 2048 // 16, _zero, 0)
    for off, sz in _PIECES:
        psl = pl.ds(sid * SLICE + off, sz)
        pltpu.sync_copy(stage_v.at[pl.ds(0, sz)], acc0_sh.at[psl])
        pltpu.sync_copy(stage_v.at[pl.ds(0, sz)], acc1_sh.at[psl])
    plsc.subcore_barrier()

    def _issue_in(g, b):
        pltpu.async_copy(edge_hbm.at[pl.ds(g * CHUNK, CHUNK)], srcb[b],
                         sem_in[b])
        pltpu.async_copy(edge_hbm.at[pl.ds(E + g * CHUNK, CHUNK)], dstb[b],
                         sem_in[b])

    def _wait_in(g, b):
        pltpu.make_async_copy(edge_hbm.at[pl.ds(g * CHUNK, CHUNK)], srcb[b],
                              sem_in[b]).wait()
        pltpu.make_async_copy(edge_hbm.at[pl.ds(E + g * CHUNK, CHUNK)],
                              dstb[b], sem_in[b]).wait()

    def _issue_sc(b):
        pltpu.async_copy(msg0[b], acc0_sh.at[dstb[b]], sem_sc[b], add=True)
        pltpu.async_copy(msg1[b], acc1_sh.at[dstb[b]], sem_sc[b], add=True)

    def _wait_sc(b):
        pltpu.make_async_copy(msg0[b], acc0_sh.at[dstb[b]], sem_sc[b]).wait()
        pltpu.make_async_copy(msg1[b], acc1_sh.at[dstb[b]], sem_sc[b]).wait()

    def _compute(b):
        for i in range(16):
            for j in range(8):
                o = i * 128 + j * 16
                s = srcb[b][pl.ds(o, 16)]
                v = plsc.load_gather(tbl_v, [s])
                m = v >= 16.0
                msg0[b][pl.ds(o, 16)] = v
                msg1[b][pl.ds(o, 16)] = jnp.where(m, 1.0, 0.0)

    _issue_in(base, 0)

    def _pair(kk2, _):
        for b in (0, 1):
            k = 2 * kk2 + b
            g = base + k

            @pl.when(g < REAL)
            def _():
                _wait_in(g, b)
                _compute(b)

            @pl.when((k >= 1) & (g - 1 < REAL))
            def _():
                _wait_sc(1 - b)

            @pl.when((k <= KPW - 2) & (g + 1 < REAL))
            def _():
                _issue_in(g + 1, 1 - b)

            @pl.when(g < REAL)
            def _():
                _issue_sc(b)
        return 0

    lax.fori_loop(0, KPW // 2, _pair, 0)

    @pl.when(base + KPW - 1 < REAL)
    def _():
        _wait_sc(1)

    plsc.subcore_barrier()

    for off, sz in _PIECES:
        psl = pl.ds(sid * SLICE + off, sz)
        osl = pl.ds(cid * NACC + sid * SLICE + off, sz)
        st = stage_v.at[pl.ds(0, sz)]
        pltpu.sync_copy(acc0_sh.at[psl], st)
        pltpu.sync_copy(st, p0_hbm.at[osl])
        pltpu.sync_copy(acc1_sh.at[psl], st)
        pltpu.sync_copy(st, p1_hbm.at[osl])


def _sc_scatter(tbl, eflat):
    mesh = plsc.VectorSubcoreMesh(core_axis_name="c", subcore_axis_name="s")
    return pl.kernel(
        _sc_body,
        out_type=(
            jax.ShapeDtypeStruct((NC * NACC,), jnp.float32),
            jax.ShapeDtypeStruct((NC * NACC,), jnp.float32),
        ),
        mesh=mesh,
        scratch_types=(
            pltpu.VMEM((TPAD,), jnp.float32),
            [pltpu.VMEM((CHUNK,), jnp.int32) for _ in range(2)],
            [pltpu.VMEM((CHUNK,), jnp.int32) for _ in range(2)],
            [pltpu.VMEM((CHUNK,), jnp.float32) for _ in range(2)],
            [pltpu.VMEM((CHUNK,), jnp.float32) for _ in range(2)],
            pltpu.VMEM((2048,), jnp.float32),
            [pltpu.SemaphoreType.DMA for _ in range(2)],
            [pltpu.SemaphoreType.DMA for _ in range(2)],
            pltpu.VMEM_SHARED((NACC,), jnp.float32),
            pltpu.VMEM_SHARED((NACC,), jnp.float32),
        ),
        compiler_params=pltpu.CompilerParams(needs_layout_passes=False),
    )(tbl, eflat)


def _merge_body(p0_ref, p1_ref, o_ref):
    vsum = p0_ref[0, :] + p0_ref[1, :]
    cnt = p1_ref[0, :] + p1_ref[1, :]
    o_ref[0, :] = vsum - 16.0 * cnt
    o_ref[1, :] = 10.0 * cnt


def _merge(p0, p1):
    return pl.pallas_call(
        _merge_body,
        out_shape=jax.ShapeDtypeStruct((2, NACC), jnp.float32),
    )(p0, p1)


@jax.jit
def kernel(belief, probs, bernoulli_uniforms, edge_index):
    pad = NPAD - N
    belief2 = jnp.pad(belief, (0, pad)).reshape(1, NPAD)
    probs2 = jnp.pad(probs, (0, pad)).reshape(1, NPAD)
    u_t = jnp.pad(bernoulli_uniforms.T, ((0, 0), (0, pad)))

    tbl = _packed_table(belief2, probs2, u_t).reshape(NPAD)

    p0, p1 = _sc_scatter(tbl, edge_index.reshape(2 * E))

    merged = _merge(p0.reshape(NC, NACC), p1.reshape(NC, NACC))
    return merged[:, :N].T
